# baseline (device time: 52496 ns/iter reference)
import os

import jax
import jax.numpy as jnp
from jax import lax
from jax.experimental import pallas as pl
from jax.experimental.pallas import tpu as pltpu

COMM = os.environ.get("SKIP_COMM", "0") != "1"

N_DEV = 8
SQ = 1024
D_MODEL = 1024
HQ = 8
DH = 128
NBLK = 16
BLK = 64
GRP = 256
CH = SQ // N_DEV
SCALE = 0.08838834764831843

PERM = [r + 4 * j for r in range(4) for j in range(4)]
IPERM = [0] * NBLK
for _i, _p in enumerate(PERM):
    IPERM[_p] = _i


def kernel(x, Wq, K_ext, V_ext, Wo):
    x2 = x.reshape(SQ, D_MODEL)
    k2 = K_ext.reshape(SQ, HQ, DH)
    v2 = V_ext.reshape(SQ, HQ, DH)

    def body(x_ref, wq_ref, k_ref, v_ref, wo_ref, out_ref,
             xp_ref, kp_ref, vp_ref, ctx_ref, acc_ref, red_ref,
             rs_buf, wq_vmem, wo_vmem,
             rs_sr, rs_rr, ag_sr, ag_rr, w_sems):
        my = lax.axis_index("i")
        rot = lax.rem(lax.div(my, 2) + 1, 4)

        def m8(v):
            return lax.rem(v + 4 * N_DEV, N_DEV)

        def chunk(ref, idx):
            return ref.at[pl.ds(idx * CH, CH), :]

        def rc(src, dst, ssem, rsem, dev):
            return pltpu.make_async_remote_copy(
                src_ref=src, dst_ref=dst, send_sem=ssem, recv_sem=rsem,
                device_id=(dev,), device_id_type=pl.DeviceIdType.MESH)

        wq_dma = pltpu.make_async_copy(
            wq_ref.at[:, pl.ds(my * (HQ * DH), HQ * DH)], wq_vmem,
            w_sems.at[0])
        wo_dma = pltpu.make_async_copy(
            wo_ref.at[pl.ds(my * (HQ * DH), HQ * DH), :], wo_vmem,
            w_sems.at[1])
        wq_dma.start()
        wo_dma.start()

        if COMM:
            bsem = pltpu.get_barrier_semaphore()
            for k in range(1, N_DEV):
                pl.semaphore_signal(bsem, inc=1, device_id=(m8(my + k),),
                                    device_id_type=pl.DeviceIdType.MESH)
            pl.semaphore_wait(bsem, N_DEV - 1)

        for j in range(NBLK):
            t, u = j // 4, j % 4
            r_g = lax.rem(t + rot, 4)
            gsrc = (r_g + 4 * u) * BLK
            xp_ref[j * BLK:(j + 1) * BLK, :] = (
                x_ref[pl.ds(gsrc, BLK), :].astype(jnp.bfloat16))

        wq_dma.wait()
        wq_bf = wq_vmem[:, :].astype(jnp.bfloat16)
        wo_dma.wait()
        wo_bf = wo_vmem[:, :].astype(jnp.bfloat16)

        for t in range(4):
            r_g = lax.rem(t + rot, 4)
            rows = slice(t * GRP, (t + 1) * GRP)
            for u in range(4):
                j = 4 * t + u
                gsrc = (r_g + 4 * u) * BLK
                kp_ref[j * BLK:(j + 1) * BLK, :, :] = (
                    k_ref[pl.ds(gsrc, BLK), :, :].astype(jnp.bfloat16))
                vp_ref[j * BLK:(j + 1) * BLK, :, :] = (
                    v_ref[pl.ds(gsrc, BLK), :, :].astype(jnp.bfloat16))

            qg = jnp.dot(xp_ref[rows, :], wq_bf,
                         preferred_element_type=jnp.float32)
            qg = qg.astype(jnp.bfloat16)

            for h in range(HQ):
                q = qg[:, h * DH:(h + 1) * DH]
                k = kp_ref[rows, h, :]
                s = jnp.dot(q, k.T,
                            preferred_element_type=jnp.float32) * SCALE
                m = jnp.max(s, axis=1, keepdims=True)
                e = jnp.exp(s - m)
                w = e / jnp.sum(e, axis=1, keepdims=True)
                ctx = jnp.dot(w.astype(jnp.bfloat16), vp_ref[rows, h, :],
                              preferred_element_type=jnp.float32)
                ctx_ref[rows, h * DH:(h + 1) * DH] = ctx.astype(jnp.bfloat16)

            for half in (0, 1):
                c = 2 * r_g + half
                acc_ref[pl.ds(c * CH, CH), :] = jnp.dot(
                    ctx_ref[t * GRP + half * CH:t * GRP + (half + 1) * CH, :],
                    wo_bf,
                    preferred_element_type=jnp.float32).astype(jnp.bfloat16)

                if COMM:
                    @pl.when(my != c)
                    def _(c=c):
                        slot = m8(my - c) - 1
                        rc(chunk(acc_ref, c), rs_buf.at[slot],
                           rs_sr.at[c], rs_rr.at[slot], c).start()

        if COMM:
            for j in range(N_DEV - 1):
                rc(rs_buf.at[j], rs_buf.at[j], rs_sr.at[0], rs_rr.at[j],
                   my).wait_recv()
            total = acc_ref[pl.ds(my * CH, CH), :].astype(jnp.float32)
            for j in range(N_DEV - 1):
                total = total + rs_buf[j, :, :].astype(jnp.float32)
            red_ref[pl.ds(my * CH, CH), :] = total.astype(jnp.bfloat16)

            ag_send = []
            for k in range(N_DEV - 1):
                dest = m8(my + 1 + k)
                d = rc(chunk(red_ref, my), chunk(red_ref, my),
                       ag_sr.at[k], ag_rr.at[N_DEV - 2 - k], dest)
                d.start()
                ag_send.append(d)
            for j in range(N_DEV - 1):
                src_dev_chunk = m8(my + 1 + j)
                rc(chunk(red_ref, my), chunk(red_ref, src_dev_chunk),
                   ag_sr.at[0], ag_rr.at[j], my).wait_recv()
            gref = red_ref
        else:
            gref = acc_ref

        for i in range(NBLK):
            src = IPERM[i] * BLK
            out_ref[i * BLK:(i + 1) * BLK, :] = gref[src:src + BLK, :]

        if COMM:
            for c in range(N_DEV):
                @pl.when(my != c)
                def _(c=c):
                    rc(chunk(acc_ref, c), rs_buf.at[0], rs_sr.at[c],
                       rs_rr.at[0], m8(my + 1)).wait_send()
            for d in ag_send:
                d.wait_send()

    out = pl.pallas_call(
        body,
        out_shape=jax.ShapeDtypeStruct((SQ, D_MODEL), jnp.bfloat16),
        in_specs=[
            pl.BlockSpec(memory_space=pltpu.VMEM),
            pl.BlockSpec(memory_space=pltpu.MemorySpace.HBM),
            pl.BlockSpec(memory_space=pltpu.VMEM),
            pl.BlockSpec(memory_space=pltpu.VMEM),
            pl.BlockSpec(memory_space=pltpu.MemorySpace.HBM),
        ],
        out_specs=pl.BlockSpec(memory_space=pltpu.VMEM),
        scratch_shapes=[
            pltpu.VMEM((SQ, D_MODEL), jnp.bfloat16),
            pltpu.VMEM((SQ, HQ, DH), jnp.bfloat16),
            pltpu.VMEM((SQ, HQ, DH), jnp.bfloat16),
            pltpu.VMEM((SQ, HQ * DH), jnp.bfloat16),
            pltpu.VMEM((SQ, D_MODEL), jnp.bfloat16),
            pltpu.VMEM((SQ, D_MODEL), jnp.bfloat16),
            pltpu.VMEM((N_DEV - 1, CH, D_MODEL), jnp.bfloat16),
            pltpu.VMEM((D_MODEL, HQ * DH), jnp.float32),
            pltpu.VMEM((HQ * DH, D_MODEL), jnp.float32),
            pltpu.SemaphoreType.DMA((N_DEV,)),
            pltpu.SemaphoreType.DMA((N_DEV - 1,)),
            pltpu.SemaphoreType.DMA((N_DEV - 1,)),
            pltpu.SemaphoreType.DMA((N_DEV - 1,)),
            pltpu.SemaphoreType.DMA((2,)),
        ],
        compiler_params=(pltpu.CompilerParams(collective_id=0)
                         if COMM else pltpu.CompilerParams()),
    )(x2, Wq, k2, v2, Wo)
    return out.reshape(1, SQ, D_MODEL)


# device time: 51803 ns/iter; 1.0134x vs baseline; 1.0134x over previous
import os

import jax
import jax.numpy as jnp
from jax import lax
from jax.experimental import pallas as pl
from jax.experimental.pallas import tpu as pltpu

COMM = os.environ.get("SKIP_COMM", "0") != "1"

N_DEV = 8
SQ = 1024
D_MODEL = 1024
HQ = 8
DH = 128
NBLK = 16
BLK = 64
GRP = 256
CH = SQ // N_DEV
SCALE = 0.08838834764831843

PERM = [r + 4 * j for r in range(4) for j in range(4)]
IPERM = [0] * NBLK
for _i, _p in enumerate(PERM):
    IPERM[_p] = _i


def kernel(x, Wq, K_ext, V_ext, Wo):
    x2 = x.reshape(SQ, D_MODEL)
    k2 = K_ext.reshape(SQ, HQ, DH)
    v2 = V_ext.reshape(SQ, HQ, DH)

    def body(x_ref, wq_ref, k_ref, v_ref, wo_ref, out_ref,
             xp_ref, kp_ref, vp_ref, ctx_ref, acc_ref, red_ref,
             rs_buf, wq_vmem, wo_vmem,
             rs_sr, rs_rr, ag_sr, ag_rr, w_sems):
        my = lax.axis_index("i")
        rot = lax.rem(lax.div(my, 2) + 1, 4)

        def m8(v):
            return lax.rem(v + 4 * N_DEV, N_DEV)

        def chunk(ref, idx):
            return ref.at[pl.ds(idx * CH, CH), :]

        def rc(src, dst, ssem, rsem, dev):
            return pltpu.make_async_remote_copy(
                src_ref=src, dst_ref=dst, send_sem=ssem, recv_sem=rsem,
                device_id=(dev,), device_id_type=pl.DeviceIdType.MESH)

        wq_dma = pltpu.make_async_copy(
            wq_ref.at[:, pl.ds(my * (HQ * DH), HQ * DH)], wq_vmem,
            w_sems.at[0])
        wo_dma = pltpu.make_async_copy(
            wo_ref.at[pl.ds(my * (HQ * DH), HQ * DH), :], wo_vmem,
            w_sems.at[1])
        wq_dma.start()
        wo_dma.start()

        if COMM:
            bsem = pltpu.get_barrier_semaphore()
            for k in range(1, N_DEV):
                pl.semaphore_signal(bsem, inc=1, device_id=(m8(my + k),),
                                    device_id_type=pl.DeviceIdType.MESH)
            pl.semaphore_wait(bsem, N_DEV - 1)

        for j in range(NBLK):
            t, u = j // 4, j % 4
            r_g = lax.rem(t + rot, 4)
            gsrc = (r_g + 4 * u) * BLK
            xp_ref[j * BLK:(j + 1) * BLK, :] = (
                x_ref[pl.ds(gsrc, BLK), :].astype(jnp.bfloat16))

        wq_dma.wait()
        wq_bf = wq_vmem[:, :].astype(jnp.bfloat16)
        wo_dma.wait()
        wo_bf = wo_vmem[:, :].astype(jnp.bfloat16)

        for t in range(4):
            r_g = lax.rem(t + rot, 4)
            rows = slice(t * GRP, (t + 1) * GRP)
            for u in range(4):
                j = 4 * t + u
                gsrc = (r_g + 4 * u) * BLK
                kp_ref[j * BLK:(j + 1) * BLK, :, :] = (
                    k_ref[pl.ds(gsrc, BLK), :, :].astype(jnp.bfloat16))
                vp_ref[j * BLK:(j + 1) * BLK, :, :] = (
                    v_ref[pl.ds(gsrc, BLK), :, :].astype(jnp.bfloat16))

            qg = jnp.dot(xp_ref[rows, :], wq_bf,
                         preferred_element_type=jnp.float32)
            qg = qg.astype(jnp.bfloat16)

            for h in range(HQ):
                q = qg[:, h * DH:(h + 1) * DH]
                k = kp_ref[rows, h, :]
                s = jnp.dot(q, k.T,
                            preferred_element_type=jnp.float32) * SCALE
                e = jnp.exp(s)
                denom = jnp.sum(e, axis=1, keepdims=True)
                ctx = jnp.dot(e.astype(jnp.bfloat16), vp_ref[rows, h, :],
                              preferred_element_type=jnp.float32)
                ctx = ctx * (1.0 / denom)
                ctx_ref[rows, h * DH:(h + 1) * DH] = ctx.astype(jnp.bfloat16)

            for half in (0, 1):
                c = 2 * r_g + half
                acc_ref[pl.ds(c * CH, CH), :] = jnp.dot(
                    ctx_ref[t * GRP + half * CH:t * GRP + (half + 1) * CH, :],
                    wo_bf,
                    preferred_element_type=jnp.float32).astype(jnp.bfloat16)

                if COMM:
                    @pl.when(my != c)
                    def _(c=c):
                        slot = m8(my - c) - 1
                        rc(chunk(acc_ref, c), rs_buf.at[slot],
                           rs_sr.at[c], rs_rr.at[slot], c).start()

        if COMM:
            for j in range(N_DEV - 1):
                rc(rs_buf.at[j], rs_buf.at[j], rs_sr.at[0], rs_rr.at[j],
                   my).wait_recv()
            total = acc_ref[pl.ds(my * CH, CH), :].astype(jnp.float32)
            for j in range(N_DEV - 1):
                total = total + rs_buf[j, :, :].astype(jnp.float32)
            red_ref[pl.ds(my * CH, CH), :] = total.astype(jnp.bfloat16)

            def unperm_store(c):
                for half in (0, 1):
                    g = 2 * c + half
                    nat = lax.div(g, 4) + 4 * lax.rem(g, 4)
                    out_ref[pl.ds(nat * BLK, BLK), :] = (
                        red_ref[pl.ds(g * BLK, BLK), :])

            ag_send = []
            for k in range(N_DEV - 1):
                dest = m8(my + 1 + k)
                d = rc(chunk(red_ref, my), chunk(red_ref, my),
                       ag_sr.at[k], ag_rr.at[N_DEV - 2 - k], dest)
                d.start()
                ag_send.append(d)
            unperm_store(my)
            for j in range(N_DEV - 1):
                src_dev_chunk = m8(my + 1 + j)
                rc(chunk(red_ref, my), chunk(red_ref, src_dev_chunk),
                   ag_sr.at[0], ag_rr.at[j], my).wait_recv()
                unperm_store(src_dev_chunk)
        else:
            for i in range(NBLK):
                src = IPERM[i] * BLK
                out_ref[i * BLK:(i + 1) * BLK, :] = acc_ref[src:src + BLK, :]

        if COMM:
            for c in range(N_DEV):
                @pl.when(my != c)
                def _(c=c):
                    rc(chunk(acc_ref, c), rs_buf.at[0], rs_sr.at[c],
                       rs_rr.at[0], m8(my + 1)).wait_send()
            for d in ag_send:
                d.wait_send()

    out = pl.pallas_call(
        body,
        out_shape=jax.ShapeDtypeStruct((SQ, D_MODEL), jnp.bfloat16),
        in_specs=[
            pl.BlockSpec(memory_space=pltpu.VMEM),
            pl.BlockSpec(memory_space=pltpu.MemorySpace.HBM),
            pl.BlockSpec(memory_space=pltpu.VMEM),
            pl.BlockSpec(memory_space=pltpu.VMEM),
            pl.BlockSpec(memory_space=pltpu.MemorySpace.HBM),
        ],
        out_specs=pl.BlockSpec(memory_space=pltpu.VMEM),
        scratch_shapes=[
            pltpu.VMEM((SQ, D_MODEL), jnp.bfloat16),
            pltpu.VMEM((SQ, HQ, DH), jnp.bfloat16),
            pltpu.VMEM((SQ, HQ, DH), jnp.bfloat16),
            pltpu.VMEM((SQ, HQ * DH), jnp.bfloat16),
            pltpu.VMEM((SQ, D_MODEL), jnp.bfloat16),
            pltpu.VMEM((SQ, D_MODEL), jnp.bfloat16),
            pltpu.VMEM((N_DEV - 1, CH, D_MODEL), jnp.bfloat16),
            pltpu.VMEM((D_MODEL, HQ * DH), jnp.float32),
            pltpu.VMEM((HQ * DH, D_MODEL), jnp.float32),
            pltpu.SemaphoreType.DMA((N_DEV,)),
            pltpu.SemaphoreType.DMA((N_DEV - 1,)),
            pltpu.SemaphoreType.DMA((N_DEV - 1,)),
            pltpu.SemaphoreType.DMA((N_DEV - 1,)),
            pltpu.SemaphoreType.DMA((2,)),
        ],
        compiler_params=(pltpu.CompilerParams(collective_id=0)
                         if COMM else pltpu.CompilerParams()),
    )(x2, Wq, k2, v2, Wo)
    return out.reshape(1, SQ, D_MODEL)


# device time: 51747 ns/iter; 1.0145x vs baseline; 1.0011x over previous
import os

import jax
import jax.numpy as jnp
from jax import lax
from jax.experimental import pallas as pl
from jax.experimental.pallas import tpu as pltpu

COMM = os.environ.get("SKIP_COMM", "0") != "1"

N_DEV = 8
SQ = 1024
D_MODEL = 1024
HQ = 8
DH = 128
NBLK = 16
BLK = 64
GRP = 256
CH = SQ // N_DEV
SCALE = 0.08838834764831843

PERM = [r + 4 * j for r in range(4) for j in range(4)]
IPERM = [0] * NBLK
for _i, _p in enumerate(PERM):
    IPERM[_p] = _i


def kernel(x, Wq, K_ext, V_ext, Wo):
    x2 = x.reshape(SQ, D_MODEL)
    k2 = K_ext.reshape(SQ, HQ, DH)
    v2 = V_ext.reshape(SQ, HQ, DH)

    def body(x_ref, wq_ref, k_ref, v_ref, wo_ref, out_ref,
             xp_ref, kp_ref, vp_ref, ctx_ref, acc_ref, red_ref,
             rs_buf, wq_vmem, wo_vmem,
             rs_sr, rs_rr, ag_sr, ag_rr, w_sems):
        my = lax.axis_index("i")
        rot = lax.rem(lax.div(my, 2) + 1, 4)

        def m8(v):
            return lax.rem(v + 4 * N_DEV, N_DEV)

        def chunk(ref, idx):
            return ref.at[pl.ds(idx * CH, CH), :]

        def rc(src, dst, ssem, rsem, dev):
            return pltpu.make_async_remote_copy(
                src_ref=src, dst_ref=dst, send_sem=ssem, recv_sem=rsem,
                device_id=(dev,), device_id_type=pl.DeviceIdType.MESH)

        wq_dma = pltpu.make_async_copy(
            wq_ref.at[:, pl.ds(my * (HQ * DH), HQ * DH)], wq_vmem,
            w_sems.at[0])
        wo_dma = pltpu.make_async_copy(
            wo_ref.at[pl.ds(my * (HQ * DH), HQ * DH), :], wo_vmem,
            w_sems.at[1])
        wq_dma.start()
        wo_dma.start()

        if COMM:
            bsem = pltpu.get_barrier_semaphore()
            for k in range(1, N_DEV):
                pl.semaphore_signal(bsem, inc=1, device_id=(m8(my + k),),
                                    device_id_type=pl.DeviceIdType.MESH)
            pl.semaphore_wait(bsem, N_DEV - 1)

        for j in range(NBLK):
            t, u = j // 4, j % 4
            r_g = lax.rem(t + rot, 4)
            gsrc = (r_g + 4 * u) * BLK
            xp_ref[j * BLK:(j + 1) * BLK, :] = (
                x_ref[pl.ds(gsrc, BLK), :].astype(jnp.bfloat16))

        wq_dma.wait()
        wq_bf = wq_vmem[:, :].astype(jnp.bfloat16)
        wo_dma.wait()
        wo_bf = wo_vmem[:, :].astype(jnp.bfloat16)

        for t in range(4):
            r_g = lax.rem(t + rot, 4)
            rows = slice(t * GRP, (t + 1) * GRP)
            for u in range(4):
                j = 4 * t + u
                gsrc = (r_g + 4 * u) * BLK
                kp_ref[j * BLK:(j + 1) * BLK, :, :] = (
                    k_ref[pl.ds(gsrc, BLK), :, :].astype(jnp.bfloat16))
                vp_ref[j * BLK:(j + 1) * BLK, :, :] = (
                    v_ref[pl.ds(gsrc, BLK), :, :].astype(jnp.bfloat16))

            qg = jnp.dot(xp_ref[rows, :], wq_bf,
                         preferred_element_type=jnp.float32)
            qg = qg.astype(jnp.bfloat16)

            for h in range(HQ):
                q = qg[:, h * DH:(h + 1) * DH]
                k = kp_ref[rows, h, :]
                s = jnp.dot(q, k.T,
                            preferred_element_type=jnp.float32)
                e = jnp.exp(s * SCALE)
                denom = jnp.sum(e, axis=1, keepdims=True)
                e_bf = e.astype(jnp.bfloat16)
                ctx = jnp.dot(e_bf, vp_ref[rows, h, :],
                              preferred_element_type=jnp.float32)
                ctx = ctx * (1.0 / denom)
                ctx_ref[rows, h * DH:(h + 1) * DH] = ctx.astype(jnp.bfloat16)

            for half in (0, 1):
                c = 2 * r_g + half
                acc_ref[pl.ds(c * CH, CH), :] = jnp.dot(
                    ctx_ref[t * GRP + half * CH:t * GRP + (half + 1) * CH, :],
                    wo_bf,
                    preferred_element_type=jnp.float32).astype(jnp.bfloat16)

                if COMM:
                    @pl.when(my != c)
                    def _(c=c):
                        slot = m8(my - c) - 1
                        rc(chunk(acc_ref, c), rs_buf.at[slot],
                           rs_sr.at[c], rs_rr.at[slot], c).start()

        if COMM:
            total = acc_ref[pl.ds(my * CH, CH), :].astype(jnp.float32)
            for j in range(N_DEV - 1):
                rc(rs_buf.at[j], rs_buf.at[j], rs_sr.at[0], rs_rr.at[j],
                   my).wait_recv()
                total = total + rs_buf[j, :, :].astype(jnp.float32)
            red_ref[pl.ds(my * CH, CH), :] = total.astype(jnp.bfloat16)

            def unperm_store(c):
                for half in (0, 1):
                    g = 2 * c + half
                    nat = lax.div(g, 4) + 4 * lax.rem(g, 4)
                    out_ref[pl.ds(nat * BLK, BLK), :] = (
                        red_ref[pl.ds(g * BLK, BLK), :])

            ag_send = []
            for k in range(N_DEV - 1):
                dest = m8(my + 1 + k)
                d = rc(chunk(red_ref, my), chunk(red_ref, my),
                       ag_sr.at[k], ag_rr.at[N_DEV - 2 - k], dest)
                d.start()
                ag_send.append(d)
            unperm_store(my)
            for j in range(N_DEV - 1):
                src_dev_chunk = m8(my + 1 + j)
                rc(chunk(red_ref, my), chunk(red_ref, src_dev_chunk),
                   ag_sr.at[0], ag_rr.at[j], my).wait_recv()
                unperm_store(src_dev_chunk)
        else:
            for i in range(NBLK):
                src = IPERM[i] * BLK
                out_ref[i * BLK:(i + 1) * BLK, :] = acc_ref[src:src + BLK, :]

        if COMM:
            for c in range(N_DEV):
                @pl.when(my != c)
                def _(c=c):
                    rc(chunk(acc_ref, c), rs_buf.at[0], rs_sr.at[c],
                       rs_rr.at[0], m8(my + 1)).wait_send()
            for d in ag_send:
                d.wait_send()

    out = pl.pallas_call(
        body,
        out_shape=jax.ShapeDtypeStruct((SQ, D_MODEL), jnp.bfloat16),
        in_specs=[
            pl.BlockSpec(memory_space=pltpu.VMEM),
            pl.BlockSpec(memory_space=pltpu.MemorySpace.HBM),
            pl.BlockSpec(memory_space=pltpu.VMEM),
            pl.BlockSpec(memory_space=pltpu.VMEM),
            pl.BlockSpec(memory_space=pltpu.MemorySpace.HBM),
        ],
        out_specs=pl.BlockSpec(memory_space=pltpu.VMEM),
        scratch_shapes=[
            pltpu.VMEM((SQ, D_MODEL), jnp.bfloat16),
            pltpu.VMEM((SQ, HQ, DH), jnp.bfloat16),
            pltpu.VMEM((SQ, HQ, DH), jnp.bfloat16),
            pltpu.VMEM((SQ, HQ * DH), jnp.bfloat16),
            pltpu.VMEM((SQ, D_MODEL), jnp.bfloat16),
            pltpu.VMEM((SQ, D_MODEL), jnp.bfloat16),
            pltpu.VMEM((N_DEV - 1, CH, D_MODEL), jnp.bfloat16),
            pltpu.VMEM((D_MODEL, HQ * DH), jnp.float32),
            pltpu.VMEM((HQ * DH, D_MODEL), jnp.float32),
            pltpu.SemaphoreType.DMA((N_DEV,)),
            pltpu.SemaphoreType.DMA((N_DEV - 1,)),
            pltpu.SemaphoreType.DMA((N_DEV - 1,)),
            pltpu.SemaphoreType.DMA((N_DEV - 1,)),
            pltpu.SemaphoreType.DMA((2,)),
        ],
        compiler_params=(pltpu.CompilerParams(collective_id=0)
                         if COMM else pltpu.CompilerParams()),
    )(x2, Wq, k2, v2, Wo)
    return out.reshape(1, SQ, D_MODEL)
